# Initial kernel scaffold; baseline (speedup 1.0000x reference)
import jax
import jax.numpy as jnp
from jax.experimental import pallas as pl


def kernel(nodes, edges, edge_index, concept_table, role_table):
    raise NotImplementedError("write your pallas kernel here")


# speedup vs baseline: 2.5978x; 2.5978x over previous
import dataclasses

import jax
import jax.numpy as jnp
from jax.experimental import pallas as pl
from jax.experimental.pallas import tpu as pltpu
from jax.experimental.pallas import tpu_sc as plsc

N_NODES = 100000
N_EDGES = 1600000
D_MODEL = 32

W_NODE = 400
W_EDGE = 800


def kernel(nodes, edges, edge_index, concept_table, role_table):
    del edge_index

    nodes2d = nodes.reshape(N_NODES // W_NODE, W_NODE).astype(jnp.int32)
    edges2d = edges.reshape(N_EDGES // W_EDGE, W_EDGE).astype(jnp.int32)

    mesh = plsc.VectorSubcoreMesh(core_axis_name="core",
                                  subcore_axis_name="subcore")

    @pl.kernel(
        out_type=jax.ShapeDtypeStruct((N_NODES + N_EDGES, D_MODEL),
                                      jnp.float32),
        mesh=mesh,
        compiler_params=dataclasses.replace(pltpu.CompilerParams(),
                                            use_tc_tiling_on_sc=False),
    )
    def sc_kernel(ct_hbm, rt_hbm, n_hbm, e_hbm, o_hbm):
        def node_body(i_vmem, o_vmem):
            pltpu.sync_copy(ct_hbm.at[i_vmem.at[0]], o_vmem)

        pltpu.emit_pipeline(
            node_body,
            grid=(N_NODES // W_NODE,),
            in_specs=[pl.BlockSpec((1, W_NODE), index_map=lambda i: (i, 0))],
            out_specs=[pl.BlockSpec((W_NODE, D_MODEL),
                                    index_map=lambda i: (i, 0))],
            core_axis_name=("core", "subcore"),
            dimension_semantics=(pltpu.PARALLEL,),
        )(n_hbm, o_hbm)

        def edge_body(i_vmem, o_vmem):
            pltpu.sync_copy(rt_hbm.at[i_vmem.at[0]], o_vmem)

        pltpu.emit_pipeline(
            edge_body,
            grid=(N_EDGES // W_EDGE,),
            in_specs=[pl.BlockSpec((1, W_EDGE), index_map=lambda i: (i, 0))],
            out_specs=[pl.BlockSpec((W_EDGE, D_MODEL),
                                    index_map=lambda i: (i + N_NODES // W_EDGE, 0))],
            core_axis_name=("core", "subcore"),
            dimension_semantics=(pltpu.PARALLEL,),
        )(e_hbm, o_hbm)

    return sc_kernel(concept_table, role_table, nodes2d, edges2d)
